# baseline (speedup 1.0000x reference)
import functools

import jax
import jax.numpy as jnp
from jax import lax
from jax.experimental import pallas as pl
from jax.experimental.pallas import tpu as pltpu
from jax.experimental.pallas import tpu_sc as plsc

NBPS = 4
CLIP = 20.0
NC, NS = 2, 16
NW = NC * NS


def _build(B, K, S):
    bpw = B // NW
    n_pi = bpw * K * S
    n_d = bpw * K
    n_out = bpw * S * NBPS
    mesh = plsc.VectorSubcoreMesh(core_axis_name="c", subcore_axis_name="s",
                                  num_cores=NC, num_subcores=NS)

    @functools.partial(
        pl.kernel,
        out_type=jax.ShapeDtypeStruct((NW, n_out), jnp.float32),
        mesh=mesh,
        scratch_types=[
            pltpu.VMEM((n_pi,), jnp.int32),
            pltpu.VMEM((n_d,), jnp.float32),
            pltpu.VMEM((n_out,), jnp.float32),
            pltpu.VMEM((24,), jnp.float32),
        ],
        compiler_params=pltpu.CompilerParams(needs_layout_passes=False),
    )
    def llr_kernel(pi_hbm, d_hbm, out_hbm, pi_v, d_v, out_v, fold_v):
        wid = lax.axis_index("s") * NC + lax.axis_index("c")
        pltpu.sync_copy(pi_hbm.at[wid], pi_v)
        pltpu.sync_copy(d_hbm.at[wid], d_v)

        iota = lax.iota(jnp.int32, 16)
        hi = iota >> 3
        lane_lt8 = iota < 8
        inf = jnp.full((16,), jnp.inf, jnp.float32)
        oidx = [(iota & 7) * NBPS + i for i in range(NBPS)]

        UNROLL = 4

        def row(b, carry):
            vbase = b * (K * S)
            dbase = b * K
            obase = b * (S * NBPS)

            def jstep(jc, accs):
                a0, a1 = list(accs[0]), list(accs[1])
                for u in range(UNROLL):
                    j = jc * UNROLL + u
                    v = pi_v[pl.ds(vbase + 16 * j, 16)]
                    dj = plsc.load_gather(d_v, [hi + (dbase + 2 * j)])
                    for i in range(NBPS):
                        m0 = (v & (8 >> i)) == 0
                        a0[i] = jnp.minimum(a0[i], jnp.where(m0, dj, inf))
                        a1[i] = jnp.minimum(a1[i], jnp.where(m0, inf, dj))
                return (tuple(a0), tuple(a1))

            a0, a1 = lax.fori_loop(0, K // 2 // UNROLL, jstep,
                                   ((inf,) * NBPS, (inf,) * NBPS))
            for i in range(NBPS):
                fold_v[pl.ds(0, 16)] = a0[i]
                f0 = jnp.minimum(a0[i], fold_v[pl.ds(8, 16)])
                fold_v[pl.ds(0, 16)] = a1[i]
                f1 = jnp.minimum(a1[i], fold_v[pl.ds(8, 16)])
                llr = jnp.clip((f0 - f1) * 0.5, -CLIP, CLIP)
                plsc.store_scatter(out_v, [obase + oidx[i]], llr,
                                   mask=lane_lt8)
            return carry

        lax.fori_loop(0, bpw, row, 0)
        pltpu.sync_copy(out_v, out_hbm.at[wid])

    return llr_kernel


def kernel(y, r, dists, path_inds, path_syms):
    B, K, S = path_inds.shape
    pi = path_inds.reshape(NW, (B // NW) * K * S)
    dd = dists.reshape(NW, (B // NW) * K)
    out = _build(B, K, S)(pi, dd)
    return out.reshape(B, S, NBPS)


# speedup vs baseline: 1.5396x; 1.5396x over previous
import functools

import jax
import jax.numpy as jnp
from jax import lax
from jax.experimental import pallas as pl
from jax.experimental.pallas import tpu as pltpu
from jax.experimental.pallas import tpu_sc as plsc

NBPS = 4
CLIP = 20.0
NC, NS = 2, 16
NW = NC * NS


def _build(B, K, S):
    bpw = B // NW
    mesh = plsc.VectorSubcoreMesh(core_axis_name="c", subcore_axis_name="s",
                                  num_cores=NC, num_subcores=NS)

    @functools.partial(
        pl.kernel,
        out_type=jax.ShapeDtypeStruct((B, S, NBPS), jnp.float32),
        mesh=mesh,
        scratch_types=[
            pltpu.VMEM((bpw, K, S), jnp.int32),
            pltpu.VMEM((bpw, K), jnp.float32),
            pltpu.VMEM((bpw, S, NBPS), jnp.float32),
            pltpu.VMEM((24,), jnp.float32),
        ],
        compiler_params=pltpu.CompilerParams(needs_layout_passes=False,
                                             use_tc_tiling_on_sc=False),
    )
    def llr_kernel(pi_hbm, d_hbm, out_hbm, pi_v, d_v, out_v, fold_v):
        wid = lax.axis_index("s") * NC + lax.axis_index("c")
        base = wid * bpw
        pltpu.sync_copy(pi_hbm.at[pl.ds(base, bpw)], pi_v)
        pltpu.sync_copy(d_hbm.at[pl.ds(base, bpw)], d_v)

        iota = lax.iota(jnp.int32, 16)
        hi = iota >> 3
        lane_s = iota & 7
        lane_lt8 = iota < 8
        inf = jnp.full((16,), jnp.inf, jnp.float32)
        splat_i = [jnp.full((16,), i, jnp.int32) for i in range(NBPS)]

        UNROLL = 4

        def row(b, carry):
            splat_b = jnp.zeros((16,), jnp.int32) + b

            def jstep(jc, accs):
                a0, a1 = list(accs[0]), list(accs[1])
                for u in range(UNROLL):
                    j = jc * UNROLL + u
                    ik = hi + 2 * j
                    v = plsc.load_gather(pi_v, [splat_b, ik, lane_s])
                    dj = plsc.load_gather(d_v, [splat_b, ik])
                    for i in range(NBPS):
                        m0 = (v & (8 >> i)) == 0
                        a0[i] = jnp.minimum(a0[i], jnp.where(m0, dj, inf))
                        a1[i] = jnp.minimum(a1[i], jnp.where(m0, inf, dj))
                return (tuple(a0), tuple(a1))

            a0, a1 = lax.fori_loop(0, K // 2 // UNROLL, jstep,
                                   ((inf,) * NBPS, (inf,) * NBPS))
            for i in range(NBPS):
                fold_v[pl.ds(0, 16)] = a0[i]
                f0 = jnp.minimum(a0[i], fold_v[pl.ds(8, 16)])
                fold_v[pl.ds(0, 16)] = a1[i]
                f1 = jnp.minimum(a1[i], fold_v[pl.ds(8, 16)])
                llr = jnp.clip((f0 - f1) * 0.5, -CLIP, CLIP)
                plsc.store_scatter(out_v, [splat_b, lane_s, splat_i[i]],
                                   llr, mask=lane_lt8)
            return carry

        lax.fori_loop(0, bpw, row, 0)
        pltpu.sync_copy(out_v, out_hbm.at[pl.ds(base, bpw)])

    return llr_kernel


def kernel(y, r, dists, path_inds, path_syms):
    B, K, S = path_inds.shape
    return _build(B, K, S)(path_inds, dists)
